# Initial kernel scaffold; baseline (speedup 1.0000x reference)
import jax
import jax.numpy as jnp
from jax.experimental import pallas as pl


def kernel(input_tokens, W, b):
    raise NotImplementedError("write your pallas kernel here")


# speedup vs baseline: 4.4613x; 4.4613x over previous
import math
import functools

import jax
import jax.numpy as jnp
from jax.experimental import pallas as pl

_CAPACITY_DISTRIBUTION = (0.125, 0.125, 0.125, 0.125, 0.125, 0.125, 0.125, 0.125)

_NEGINF_KEY = -2139095041
_INT32_MIN = -2147483648


def _probs_kernel(x_ref, w_ref, b_ref, out_ref):
    x = x_ref[...]
    w = w_ref[...]
    logits = jax.lax.dot_general(
        w, x, (((1,), (1,)), ((), ())),
        precision=jax.lax.Precision.DEFAULT,
        preferred_element_type=jnp.float32)
    logits = logits + b_ref[...]
    m = jnp.max(logits, axis=0, keepdims=True)
    e = jnp.exp(logits - m)
    out_ref[...] = e / jnp.sum(e, axis=0, keepdims=True)


def _routing_kernel(probs_ref, mask_ref, ep_ref, *, caps):
    E, B, T = probs_ref.shape
    iota = jax.lax.broadcasted_iota(jnp.int32, (B, T), 1)
    maskv = jnp.full((1, T), -1, jnp.int32)

    for j in reversed(range(E)):
        cap = caps[j]
        if cap == 0:
            continue
        p = probs_ref[j]
        bits = jax.lax.bitcast_convert_type(p, jnp.int32)
        key = bits ^ ((bits >> 31) & jnp.int32(0x7FFFFFFF))
        k = jnp.where(maskv != -1, jnp.int32(_NEGINF_KEY), key)

        cnt0 = jnp.sum((k >= 0).astype(jnp.int32), axis=1, keepdims=True)
        base = jnp.where(cnt0 >= cap, jnp.int32(0), jnp.int32(_INT32_MIN))

        def vbody(i, base):
            bit = 30 - i
            trial = base + (jnp.int32(1) << bit)
            cnt = jnp.sum((k >= trial).astype(jnp.int32), axis=1, keepdims=True)
            return jnp.where(cnt >= cap, trial, base)

        vstar = jax.lax.fori_loop(0, 31, vbody, base)

        gt = k > vstar
        c_gt = jnp.sum(gt.astype(jnp.int32), axis=1, keepdims=True)
        need = cap - c_gt
        eq = k == vstar

        def ibody(i, basei):
            bit = 12 - i
            trial = basei + (jnp.int32(1) << bit)
            cnt = jnp.sum((eq & (iota < trial)).astype(jnp.int32),
                          axis=1, keepdims=True)
            return jnp.where(cnt < need, trial, basei)

        ibase = jax.lax.fori_loop(0, 13, ibody,
                                  jnp.zeros((B, 1), jnp.int32))
        sel = gt | (eq & (iota < (ibase + 1)))
        sel_any = jnp.max(sel.astype(jnp.int32), axis=0, keepdims=True)
        maskv = jnp.where(sel_any > 0, jnp.int32(j), maskv)

    maskv = jnp.where(maskv == -1, 0, maskv)
    mask_ref[...] = jnp.broadcast_to(maskv, (B, T))
    ep = jnp.zeros((B, T), jnp.float32)
    for e in range(E):
        ep = ep + probs_ref[e] * (maskv == e).astype(jnp.float32)
    ep_ref[...] = ep


def kernel(input_tokens, W, b):
    B, T, D = input_tokens.shape
    E = W.shape[0]
    caps = tuple(int(math.floor(_CAPACITY_DISTRIBUTION[j] * T)) for j in range(E))

    x = input_tokens.reshape(B * T, D)
    TILE = 2048
    ntiles = (B * T) // TILE

    probs = pl.pallas_call(
        _probs_kernel,
        grid=(ntiles,),
        in_specs=[
            pl.BlockSpec((TILE, D), lambda i: (i, 0)),
            pl.BlockSpec((E, D), lambda i: (0, 0)),
            pl.BlockSpec((E, 1), lambda i: (0, 0)),
        ],
        out_specs=pl.BlockSpec((E, TILE), lambda i: (0, i)),
        out_shape=jax.ShapeDtypeStruct((E, B * T), jnp.float32),
    )(x, W, b.reshape(E, 1))

    probs = probs.reshape(E, B, T)
    mask, ep = pl.pallas_call(
        functools.partial(_routing_kernel, caps=caps),
        out_shape=(
            jax.ShapeDtypeStruct((B, T), jnp.int32),
            jax.ShapeDtypeStruct((B, T), jnp.float32),
        ),
    )(probs)
    return (mask, ep)


# speedup vs baseline: 6.6755x; 1.4963x over previous
import math
import functools

import jax
import jax.numpy as jnp
from jax.experimental import pallas as pl

_CAPACITY_DISTRIBUTION = (0.125, 0.125, 0.125, 0.125, 0.125, 0.125, 0.125, 0.125)

_NEGINF_KEY = -2139095041


def _probs_kernel(x_ref, w_ref, b_ref, out_ref):
    x = x_ref[...]
    w = w_ref[...]
    logits = jax.lax.dot_general(
        w, x, (((1,), (1,)), ((), ())),
        precision=jax.lax.Precision.DEFAULT,
        preferred_element_type=jnp.float32)
    logits = logits + b_ref[...]
    m = jnp.max(logits, axis=0, keepdims=True)
    e = jnp.exp(logits - m)
    out_ref[...] = e / jnp.sum(e, axis=0, keepdims=True)


def _csum(x):
    return jnp.sum(x.astype(jnp.int32), axis=(1, 2), keepdims=True)


def _routing_kernel(probs_ref, mask_ref, ep_ref, *, caps):
    E, B, S, L = probs_ref.shape
    T = S * L
    iota = (jax.lax.broadcasted_iota(jnp.int32, (1, S, L), 1) * L
            + jax.lax.broadcasted_iota(jnp.int32, (1, S, L), 2))
    maskv = jnp.full((1, S, L), -1, jnp.int32)

    for j in reversed(range(E)):
        cap = caps[j]
        if cap == 0:
            continue
        assigned = maskv != -1
        avail = T - jnp.sum(assigned.astype(jnp.int32))
        p = probs_ref[j]
        bits = jax.lax.bitcast_convert_type(p, jnp.int32)
        key = bits ^ ((bits >> 31) & jnp.int32(0x7FFFFFFF))
        k = jnp.where(assigned, jnp.int32(_NEGINF_KEY), key)

        def degenerate(k):
            need = cap - avail

            def ibody(i, basei):
                trial = basei + (jnp.int32(1) << (12 - i))
                cnt = jnp.sum((assigned & (iota < trial)).astype(jnp.int32))
                return jnp.where(cnt < need, trial, basei)

            ibase = jax.lax.fori_loop(0, 13, ibody, jnp.int32(0))
            sel = (~assigned) | (assigned & (iota < (ibase + 1)))
            return sel.astype(jnp.int32)

        def search(k):
            def vbody(i, base):
                trial = base + (jnp.int32(1) << (29 - i))
                cnt = _csum(k >= trial)
                return jnp.where(cnt >= cap, trial, base)

            vstar = jax.lax.fori_loop(0, 30, vbody,
                                      jnp.zeros((B, 1, 1), jnp.int32))
            gt = k > vstar
            eq = k == vstar
            c_gt = _csum(gt)
            need = cap - c_gt
            c_eq = _csum(eq)

            def no_ties(_):
                return (gt | eq).astype(jnp.int32)

            def ties(_):
                def ibody(i, basei):
                    trial = basei + (jnp.int32(1) << (12 - i))
                    cnt = _csum(eq & (iota < trial))
                    return jnp.where(cnt < need, trial, basei)

                ibase = jax.lax.fori_loop(0, 13, ibody,
                                          jnp.zeros((B, 1, 1), jnp.int32))
                return (gt | (eq & (iota < (ibase + 1)))).astype(jnp.int32)

            sel = jax.lax.cond(
                jnp.sum((c_eq == need).astype(jnp.int32)) == B,
                no_ties, ties, 0)
            return jnp.max(sel, axis=0, keepdims=True)

        sel_any = jax.lax.cond(avail < cap, degenerate, search, k)
        maskv = jnp.where(sel_any > 0, jnp.int32(j), maskv)

    maskv = jnp.where(maskv == -1, 0, maskv)
    mask_ref[...] = jnp.broadcast_to(maskv, (B, S, L))
    ep = jnp.zeros((B, S, L), jnp.float32)
    for e in range(E):
        ep = ep + probs_ref[e] * (maskv == e).astype(jnp.float32)
    ep_ref[...] = ep


def kernel(input_tokens, W, b):
    B, T, D = input_tokens.shape
    E = W.shape[0]
    caps = tuple(int(math.floor(_CAPACITY_DISTRIBUTION[j] * T)) for j in range(E))

    x = input_tokens.reshape(B * T, D)
    TILE = 4096
    ntiles = (B * T) // TILE

    probs = pl.pallas_call(
        _probs_kernel,
        grid=(ntiles,),
        in_specs=[
            pl.BlockSpec((TILE, D), lambda i: (i, 0)),
            pl.BlockSpec((E, D), lambda i: (0, 0)),
            pl.BlockSpec((E, 1), lambda i: (0, 0)),
        ],
        out_specs=pl.BlockSpec((E, TILE), lambda i: (0, i)),
        out_shape=jax.ShapeDtypeStruct((E, B * T), jnp.float32),
    )(x, W, b.reshape(E, 1))

    S = 8
    probs = probs.reshape(E, B, S, T // S)
    mask, ep = pl.pallas_call(
        functools.partial(_routing_kernel, caps=caps),
        out_shape=(
            jax.ShapeDtypeStruct((B, S, T // S), jnp.int32),
            jax.ShapeDtypeStruct((B, S, T // S), jnp.float32),
        ),
    )(probs)
    return (mask.reshape(B, T), ep.reshape(B, T))


# speedup vs baseline: 6.9696x; 1.0441x over previous
import math
import functools

import jax
import jax.numpy as jnp
from jax.experimental import pallas as pl

_CAPACITY_DISTRIBUTION = (0.125, 0.125, 0.125, 0.125, 0.125, 0.125, 0.125, 0.125)

_NEGINF_KEY = -2139095041


def _probs_kernel(x_ref, w_ref, b_ref, out_ref):
    x = x_ref[...]
    w = w_ref[...]
    logits = jax.lax.dot_general(
        w, x, (((1,), (1,)), ((), ())),
        precision=jax.lax.Precision.DEFAULT,
        preferred_element_type=jnp.float32)
    logits = logits + b_ref[...]
    m = jnp.max(logits, axis=0, keepdims=True)
    e = jnp.exp(logits - m)
    out_ref[...] = e / jnp.sum(e, axis=0, keepdims=True)


def _csum(x):
    return jnp.sum(x.astype(jnp.int32), axis=(1, 2), keepdims=True)


def _routing_kernel(probs_ref, mask_ref, ep_ref, *, caps):
    E, B, S, L = probs_ref.shape
    T = S * L
    iota = (jax.lax.broadcasted_iota(jnp.int32, (1, S, L), 1) * L
            + jax.lax.broadcasted_iota(jnp.int32, (1, S, L), 2))
    maskv = jnp.full((1, S, L), -1, jnp.int32)

    for j in reversed(range(E)):
        cap = caps[j]
        if cap == 0:
            continue
        assigned = maskv != -1
        avail = T - jnp.sum(assigned.astype(jnp.int32))
        p = probs_ref[j]
        bits = jax.lax.bitcast_convert_type(p, jnp.int32)
        key = bits ^ ((bits >> 31) & jnp.int32(0x7FFFFFFF))

        def degenerate(_):
            need = cap - avail

            def ibody(i, basei):
                trial = basei + (jnp.int32(1) << (12 - i))
                cnt = jnp.sum((assigned & (iota < trial)).astype(jnp.int32))
                return jnp.where(cnt < need, trial, basei)

            ibase = jax.lax.fori_loop(0, 13, ibody, jnp.int32(0))
            sel = (~assigned) | (assigned & (iota < (ibase + 1)))
            return sel.astype(jnp.int32)

        def search(_):
            ks = jnp.where(assigned, jnp.int32(-1), key)

            def vbody(i, base):
                shift = 27 - 3 * i
                u = (ks - base) >> shift
                msel = jnp.zeros((B, 1, 1), jnp.int32)
                for m in range(1, 8):
                    msel = msel + (_csum(u >= m) >= cap).astype(jnp.int32)
                return base + (msel << shift)

            vstar = jax.lax.fori_loop(0, 10, vbody,
                                      jnp.zeros((B, 1, 1), jnp.int32))
            k = ks
            gt = k > vstar
            eq = k == vstar
            c_gt = _csum(gt)
            need = cap - c_gt
            c_eq = _csum(eq)

            def no_ties(_):
                return (gt | eq).astype(jnp.int32)

            def ties(_):
                def ibody(i, basei):
                    trial = basei + (jnp.int32(1) << (12 - i))
                    cnt = _csum(eq & (iota < trial))
                    return jnp.where(cnt < need, trial, basei)

                ibase = jax.lax.fori_loop(0, 13, ibody,
                                          jnp.zeros((B, 1, 1), jnp.int32))
                return (gt | (eq & (iota < (ibase + 1)))).astype(jnp.int32)

            sel = jax.lax.cond(
                jnp.sum((c_eq == need).astype(jnp.int32)) == B,
                no_ties, ties, 0)
            return jnp.max(sel, axis=0, keepdims=True)

        sel_any = jax.lax.cond(avail < cap, degenerate, search, 0)
        maskv = jnp.where(sel_any > 0, jnp.int32(j), maskv)

    maskv = jnp.where(maskv == -1, 0, maskv)
    mask_ref[...] = jnp.broadcast_to(maskv, (B, S, L))
    ep = jnp.zeros((B, S, L), jnp.float32)
    for e in range(E):
        ep = ep + probs_ref[e] * (maskv == e).astype(jnp.float32)
    ep_ref[...] = ep


def kernel(input_tokens, W, b):
    B, T, D = input_tokens.shape
    E = W.shape[0]
    caps = tuple(int(math.floor(_CAPACITY_DISTRIBUTION[j] * T)) for j in range(E))

    x = input_tokens.reshape(B * T, D)
    TILE = 4096
    ntiles = (B * T) // TILE

    probs = pl.pallas_call(
        _probs_kernel,
        grid=(ntiles,),
        in_specs=[
            pl.BlockSpec((TILE, D), lambda i: (i, 0)),
            pl.BlockSpec((E, D), lambda i: (0, 0)),
            pl.BlockSpec((E, 1), lambda i: (0, 0)),
        ],
        out_specs=pl.BlockSpec((E, TILE), lambda i: (0, i)),
        out_shape=jax.ShapeDtypeStruct((E, B * T), jnp.float32),
    )(x, W, b.reshape(E, 1))

    S = 8
    probs = probs.reshape(E, B, S, T // S)
    mask, ep = pl.pallas_call(
        functools.partial(_routing_kernel, caps=caps),
        out_shape=(
            jax.ShapeDtypeStruct((B, S, T // S), jnp.int32),
            jax.ShapeDtypeStruct((B, S, T // S), jnp.float32),
        ),
    )(probs)
    return (mask.reshape(B, T), ep.reshape(B, T))


# speedup vs baseline: 7.1883x; 1.0314x over previous
import math
import functools

import jax
import jax.numpy as jnp
from jax.experimental import pallas as pl

_CAPACITY_DISTRIBUTION = (0.125, 0.125, 0.125, 0.125, 0.125, 0.125, 0.125, 0.125)

_NEGINF_KEY = -2139095041


def _probs_kernel(x_ref, w_ref, b_ref, out_ref):
    x = x_ref[...]
    w = w_ref[...]
    logits = jax.lax.dot_general(
        w, x, (((1,), (1,)), ((), ())),
        precision=jax.lax.Precision.DEFAULT,
        preferred_element_type=jnp.float32)
    logits = logits + b_ref[...]
    m = jnp.max(logits, axis=0, keepdims=True)
    e = jnp.exp(logits - m)
    out_ref[...] = e / jnp.sum(e, axis=0, keepdims=True)


def _csum(x):
    return jnp.sum(jnp.where(x, 1.0, 0.0), axis=(1, 2), keepdims=True)


def _routing_kernel(probs_ref, mask_ref, ep_ref, *, caps):
    E, B, S, L = probs_ref.shape
    T = S * L
    iota = (jax.lax.broadcasted_iota(jnp.int32, (1, S, L), 1) * L
            + jax.lax.broadcasted_iota(jnp.int32, (1, S, L), 2))
    maskv = jnp.full((1, S, L), -1, jnp.int32)

    for j in reversed(range(E)):
        cap = caps[j]
        if cap == 0:
            continue
        assigned = maskv != -1
        avail = T - jnp.sum(jnp.where(assigned, 1.0, 0.0))
        p = probs_ref[j]
        bits = jax.lax.bitcast_convert_type(p, jnp.int32)
        key = bits ^ ((bits >> 31) & jnp.int32(0x7FFFFFFF))

        def degenerate(_):
            need = cap - avail

            basei = jnp.int32(0)
            for i in range(13):
                trial = basei + (1 << (12 - i))
                cnt = jnp.sum(jnp.where(assigned & (iota < trial), 1.0, 0.0))
                basei = jnp.where(cnt < need, trial, basei)
            sel = (~assigned) | (assigned & (iota < (basei + 1)))
            return sel.astype(jnp.int32)

        def search(_):
            ks = jnp.where(assigned, jnp.int32(-1), key)
            fcap = float(cap)

            base = jnp.zeros((B, 1, 1), jnp.int32)
            for i in range(10):
                shift = 27 - 3 * i
                u = (ks - base) >> shift
                msel = jnp.zeros((B, 1, 1), jnp.float32)
                for m in range(1, 8):
                    msel = msel + jnp.where(_csum(u >= m) >= fcap, 1.0, 0.0)
                base = base + (msel.astype(jnp.int32) << shift)

            vstar = base
            gt = ks > vstar
            eq = ks == vstar
            c_gt = _csum(gt)
            need = fcap - c_gt
            c_eq = _csum(eq)

            def no_ties(_):
                return (gt | eq).astype(jnp.int32)

            def ties(_):
                basei = jnp.zeros((B, 1, 1), jnp.int32)
                for i in range(13):
                    trial = basei + (1 << (12 - i))
                    cnt = _csum(eq & (iota < trial))
                    basei = jnp.where(cnt < need, trial, basei)
                return (gt | (eq & (iota < (basei + 1)))).astype(jnp.int32)

            sel = jax.lax.cond(
                jnp.sum(jnp.where(c_eq == need, 1.0, 0.0)) == float(B),
                no_ties, ties, 0)
            return jnp.max(sel, axis=0, keepdims=True)

        sel_any = jax.lax.cond(avail < cap, degenerate, search, 0)
        maskv = jnp.where(sel_any > 0, jnp.int32(j), maskv)

    maskv = jnp.where(maskv == -1, 0, maskv)
    mask_ref[...] = jnp.broadcast_to(maskv, (B, S, L))
    ep = jnp.zeros((B, S, L), jnp.float32)
    for e in range(E):
        ep = ep + probs_ref[e] * (maskv == e).astype(jnp.float32)
    ep_ref[...] = ep


def kernel(input_tokens, W, b):
    B, T, D = input_tokens.shape
    E = W.shape[0]
    caps = tuple(int(math.floor(_CAPACITY_DISTRIBUTION[j] * T)) for j in range(E))

    x = input_tokens.reshape(B * T, D)
    TILE = 4096
    ntiles = (B * T) // TILE

    probs = pl.pallas_call(
        _probs_kernel,
        grid=(ntiles,),
        in_specs=[
            pl.BlockSpec((TILE, D), lambda i: (i, 0)),
            pl.BlockSpec((E, D), lambda i: (0, 0)),
            pl.BlockSpec((E, 1), lambda i: (0, 0)),
        ],
        out_specs=pl.BlockSpec((E, TILE), lambda i: (0, i)),
        out_shape=jax.ShapeDtypeStruct((E, B * T), jnp.float32),
    )(x, W, b.reshape(E, 1))

    S = 8
    probs = probs.reshape(E, B, S, T // S)
    mask, ep = pl.pallas_call(
        functools.partial(_routing_kernel, caps=caps),
        out_shape=(
            jax.ShapeDtypeStruct((B, S, T // S), jnp.int32),
            jax.ShapeDtypeStruct((B, S, T // S), jnp.float32),
        ),
    )(probs)
    return (mask.reshape(B, T), ep.reshape(B, T))


# speedup vs baseline: 8.2300x; 1.1449x over previous
import math
import functools

import jax
import jax.numpy as jnp
from jax.experimental import pallas as pl

_CAPACITY_DISTRIBUTION = (0.125, 0.125, 0.125, 0.125, 0.125, 0.125, 0.125, 0.125)

_NEGINF_KEY = -2139095041


def _probs_kernel(x_ref, w_ref, b_ref, out_ref):
    x = x_ref[...]
    w = w_ref[...]
    logits = jax.lax.dot_general(
        w, x, (((1,), (1,)), ((), ())),
        precision=jax.lax.Precision.DEFAULT,
        preferred_element_type=jnp.float32)
    logits = logits + b_ref[...]
    m = jnp.max(logits, axis=0, keepdims=True)
    e = jnp.exp(logits - m)
    out_ref[...] = e / jnp.sum(e, axis=0, keepdims=True)


def _csum(x):
    return jnp.sum(jnp.where(x, 1.0, 0.0), axis=(1, 2), keepdims=True)


def _routing_kernel(probs_ref, mask_ref, ep_ref, *, caps):
    E, B, S, L = probs_ref.shape
    T = S * L
    iota = (jax.lax.broadcasted_iota(jnp.int32, (1, S, L), 1) * L
            + jax.lax.broadcasted_iota(jnp.int32, (1, S, L), 2))
    maskv = jnp.full((1, S, L), -1, jnp.int32)

    for j in reversed(range(E)):
        cap = caps[j]
        if cap == 0:
            continue
        assigned = maskv != -1
        avail = T - jnp.sum(jnp.where(assigned, 1.0, 0.0))
        p = probs_ref[j]
        bits = jax.lax.bitcast_convert_type(p, jnp.int32)
        key = bits ^ ((bits >> 31) & jnp.int32(0x7FFFFFFF))

        def degenerate(_):
            need = cap - avail

            def all_assigned(_):
                return (iota < cap).astype(jnp.int32)

            def partial(_):
                basei = jnp.int32(0)
                for shift in (12, 9, 6, 3, 0):
                    msel = jnp.float32(0.0)
                    for m in range(1, 8):
                        trial = basei + (m << shift)
                        cnt = jnp.sum(
                            jnp.where(assigned & (iota < trial), 1.0, 0.0))
                        msel = msel + jnp.where(cnt < need, 1.0, 0.0)
                    basei = basei + (msel.astype(jnp.int32) << shift)
                sel = (~assigned) | (assigned & (iota < (basei + 1)))
                return sel.astype(jnp.int32)

            return jax.lax.cond(avail == 0.0, all_assigned, partial, 0)

        def search(_):
            ks = jnp.where(assigned, jnp.int32(-1), key)
            fcap = float(cap)

            base = jnp.zeros((B, 1, 1), jnp.int32)
            for i in range(10):
                shift = 27 - 3 * i
                u = (ks - base) >> shift
                msel = jnp.zeros((B, 1, 1), jnp.float32)
                for m in range(1, 8):
                    msel = msel + jnp.where(_csum(u >= m) >= fcap, 1.0, 0.0)
                base = base + (msel.astype(jnp.int32) << shift)

            vstar = base
            gt = ks > vstar
            eq = ks == vstar
            c_gt = _csum(gt)
            need = fcap - c_gt
            c_eq = _csum(eq)

            def no_ties(_):
                return (gt | eq).astype(jnp.int32)

            def ties(_):
                basei = jnp.zeros((B, 1, 1), jnp.int32)
                for i in range(13):
                    trial = basei + (1 << (12 - i))
                    cnt = _csum(eq & (iota < trial))
                    basei = jnp.where(cnt < need, trial, basei)
                return (gt | (eq & (iota < (basei + 1)))).astype(jnp.int32)

            sel = jax.lax.cond(
                jnp.sum(jnp.where(c_eq == need, 1.0, 0.0)) == float(B),
                no_ties, ties, 0)
            return jnp.max(sel, axis=0, keepdims=True)

        sel_any = jax.lax.cond(avail < cap, degenerate, search, 0)
        maskv = jnp.where(sel_any > 0, jnp.int32(j), maskv)

    maskv = jnp.where(maskv == -1, 0, maskv)
    mask_ref[...] = jnp.broadcast_to(maskv, (B, S, L))
    ep = jnp.zeros((B, S, L), jnp.float32)
    for e in range(E):
        ep = ep + probs_ref[e] * (maskv == e).astype(jnp.float32)
    ep_ref[...] = ep


def kernel(input_tokens, W, b):
    B, T, D = input_tokens.shape
    E = W.shape[0]
    caps = tuple(int(math.floor(_CAPACITY_DISTRIBUTION[j] * T)) for j in range(E))

    x = input_tokens.reshape(B * T, D)
    TILE = 4096
    ntiles = (B * T) // TILE

    probs = pl.pallas_call(
        _probs_kernel,
        grid=(ntiles,),
        in_specs=[
            pl.BlockSpec((TILE, D), lambda i: (i, 0)),
            pl.BlockSpec((E, D), lambda i: (0, 0)),
            pl.BlockSpec((E, 1), lambda i: (0, 0)),
        ],
        out_specs=pl.BlockSpec((E, TILE), lambda i: (0, i)),
        out_shape=jax.ShapeDtypeStruct((E, B * T), jnp.float32),
    )(x, W, b.reshape(E, 1))

    S = 8
    probs = probs.reshape(E, B, S, T // S)
    mask, ep = pl.pallas_call(
        functools.partial(_routing_kernel, caps=caps),
        out_shape=(
            jax.ShapeDtypeStruct((B, S, T // S), jnp.int32),
            jax.ShapeDtypeStruct((B, S, T // S), jnp.float32),
        ),
    )(probs)
    return (mask.reshape(B, T), ep.reshape(B, T))
